# baseline (speedup 1.0000x reference)
import functools

import jax
import jax.numpy as jnp
from jax import lax
from jax.experimental import pallas as pl
from jax.experimental.pallas import tpu as pltpu
from jax.experimental.pallas import tpu_sc as plsc

N = 10000
M = 5000
P = 320000
D = 128

NC, NS = 2, 16
NW = NC * NS
PPW = P // NW
C = 80
NCHUNK = PPW // C

MP = 5120
NP = 10240
MROWS = MP // NS
NROWS = NP // NS

F32 = jnp.float32

_WIN = [i * 16 for i in range(C // 16)]


@functools.cache
def _mesh():
    return plsc.VectorSubcoreMesh(
        core_axis_name="c", subcore_axis_name="s",
        num_cores=NC, num_subcores=NS)


@functools.cache
def _make_seg(acc_rows, with_counts=False):
    assert (NCHUNK - 2) % 3 == 0
    rows_per = acc_rows // NS

    out_types = [jax.ShapeDtypeStruct((NC * acc_rows, D), F32)]
    scratch = (
        [pltpu.VMEM_SHARED((acc_rows, D), F32)]
        + [pltpu.VMEM((2 * C,), jnp.int32) for _ in range(3)]
        + [pltpu.VMEM((C,), jnp.int32) for _ in range(3)]
        + [pltpu.VMEM((C, D), F32) for _ in range(3)]
        + [pltpu.SemaphoreType.DMA for _ in range(9)]
    )
    if with_counts:
        out_types += [jax.ShapeDtypeStruct((NC * MP,), F32),
                      jax.ShapeDtypeStruct((NC * NP,), F32)]
        scratch += (
            [pltpu.VMEM_SHARED((MP,), F32),
             pltpu.VMEM_SHARED((NP,), F32)]
            + [pltpu.VMEM((C,), jnp.int32) for _ in range(3)]
            + [pltpu.VMEM((C,), F32),
               pltpu.VMEM((NROWS,), F32)]
        )

    def body_fn(*refs):
        if with_counts:
            (tab, iv, zrow, out, out_ec, out_vc, acc,
             b0, b1, b2, s0, s1, s2, r0, r1, r2,
             si0, si1, si2, sg0, sg1, sg2, ss0, ss1, ss2,
             ecnt, vcnt, g0, g1, g2, ones_v, cnt_v) = refs
            gf = [g0, g1, g2]
        else:
            (tab, iv, zrow, out, acc,
             b0, b1, b2, s0, s1, s2, r0, r1, r2,
             si0, si1, si2, sg0, sg1, sg2, ss0, ss1, ss2) = refs
            gf = [None, None, None]
        buf = [b0, b1, b2]
        sf = [s0, s1, s2]
        rows = [r0, r1, r2]
        semi = [si0, si1, si2]
        semg = [sg0, sg1, sg2]
        sems = [ss0, ss1, ss2]
        cid = lax.axis_index("c")
        sid = lax.axis_index("s")
        wid = sid * NC + cid
        cbase = wid * NCHUNK

        for r in range(rows_per // MROWS):
            pltpu.sync_copy(
                zrow.at[pl.ds(0, MROWS), :],
                acc.at[pl.ds(sid * rows_per + r * MROWS, MROWS), :])
        if with_counts:
            for i in range(NROWS // 16):
                cnt_v[pl.ds(i * 16, 16)] = jnp.zeros((16,), F32)
            pltpu.sync_copy(cnt_v.at[pl.ds(0, MROWS)],
                            ecnt.at[pl.ds(sid * MROWS, MROWS)])
            pltpu.sync_copy(cnt_v, vcnt.at[pl.ds(sid * NROWS, NROWS)])
            for i in range(C // 16):
                ones_v[pl.ds(i * 16, 16)] = jnp.ones((16,), F32)
        plsc.subcore_barrier()

        def idx_start(j, s):
            pltpu.async_copy(iv.at[pl.ds((cbase + j) * 2 * C, 2 * C)],
                             buf[s], semi[s])

        def idx_wait(s):
            pltpu.make_async_copy(iv.at[pl.ds(0, 2 * C)], buf[s],
                                  semi[s]).wait()

        def extract(s):
            for w in _WIN:
                sf[s][pl.ds(w, 16)] = buf[s][pl.ds(C + w, 16)]
            if with_counts:
                for w in _WIN:
                    gf[s][pl.ds(w, 16)] = buf[s][pl.ds(w, 16)]

        def gather_start(s):
            pltpu.async_copy(tab.at[buf[s].at[pl.ds(0, C)]], rows[s],
                             semg[s])

        def gather_wait(s):
            pltpu.make_async_copy(tab.at[buf[s].at[pl.ds(0, C)]], rows[s],
                                  semg[s]).wait()

        def scatter_start(s):
            pltpu.async_copy(rows[s], acc.at[sf[s]], sems[s], add=True)
            if with_counts:
                pltpu.sync_copy(ones_v, ecnt.at[sf[s]], add=True)
                pltpu.sync_copy(ones_v, vcnt.at[gf[s]], add=True)

        def scatter_wait(s):
            pltpu.make_async_copy(rows[s], acc.at[sf[s]], sems[s]).wait()

        idx_start(0, 0)
        idx_start(1, 1)
        idx_start(2, 2)
        idx_wait(0)
        gather_start(0)
        idx_wait(1)
        gather_start(1)
        gather_wait(0)
        extract(0)
        idx_start(3, 0)
        scatter_start(0)

        def ring(k, carry):
            idx_wait(2)

            @pl.when(k > 0)
            def _():
                scatter_wait(2)

            gather_start(2)
            gather_wait(1)
            extract(1)
            idx_start(3 * k + 4, 1)
            scatter_start(1)
            idx_wait(0)
            scatter_wait(0)
            gather_start(0)
            gather_wait(2)
            extract(2)

            @pl.when(k < (NCHUNK - 5) // 3)
            def _():
                idx_start(3 * k + 5, 2)

            scatter_start(2)
            idx_wait(1)
            scatter_wait(1)
            gather_start(1)
            gather_wait(0)
            extract(0)

            @pl.when(k < (NCHUNK - 5) // 3)
            def _():
                idx_start(3 * k + 6, 0)

            scatter_start(0)
            return carry

        lax.fori_loop(0, (NCHUNK - 2) // 3, ring, 0)
        gather_wait(1)
        extract(1)
        scatter_start(1)
        scatter_wait(0)
        scatter_wait(1)
        scatter_wait(2)

        plsc.subcore_barrier()
        pltpu.sync_copy(
            acc.at[pl.ds(sid * rows_per, rows_per), :],
            out.at[pl.ds(cid * acc_rows + sid * rows_per, rows_per), :])
        if with_counts:
            pltpu.sync_copy(ecnt.at[pl.ds(sid * MROWS, MROWS)],
                            cnt_v.at[pl.ds(0, MROWS)])
            pltpu.sync_copy(cnt_v.at[pl.ds(0, MROWS)],
                            out_ec.at[pl.ds(cid * MP + sid * MROWS, MROWS)])
            pltpu.sync_copy(vcnt.at[pl.ds(sid * NROWS, NROWS)], cnt_v)
            pltpu.sync_copy(cnt_v,
                            out_vc.at[pl.ds(cid * NP + sid * NROWS, NROWS)])

    return functools.partial(
        pl.kernel,
        mesh=_mesh(),
        out_type=tuple(out_types) if with_counts else out_types[0],
        scratch_types=scratch,
    )(body_fn)


def _theta(x, w, b):
    def body(x_ref, w_ref, b_ref, o_ref):
        o_ref[...] = jnp.dot(x_ref[...], w_ref[...],
                             preferred_element_type=F32) + b_ref[...]
    return pl.pallas_call(
        body, out_shape=jax.ShapeDtypeStruct(x.shape, F32),
    )(x, w, b.reshape(1, D))


def _pair(ref, rows, pad_rows):
    return (ref[pl.ds(0, rows), :], ref[pl.ds(pad_rows, rows), :])


def _combine_first(ep, ec0, ec1, vc0, vc1):
    def body(ep_ref, e0_ref, e1_ref, v0_ref, v1_ref,
             eo_ref, ie_ref, iv_ref):
        ie = 1.0 / jnp.maximum(e0_ref[...] + e1_ref[...], 1.0)
        iv = 1.0 / jnp.maximum(v0_ref[...] + v1_ref[...], 1.0)
        a, b = _pair(ep_ref, M, MP)
        eo_ref[...] = (a + b) * ie
        ie_ref[...] = ie
        iv_ref[...] = iv
    return pl.pallas_call(
        body,
        out_shape=(
            jax.ShapeDtypeStruct((M, D), F32),
            jax.ShapeDtypeStruct((M, 1), F32),
            jax.ShapeDtypeStruct((N, 1), F32),
        ),
    )(ep, ec0, ec1, vc0, vc1)


def _combine_scale(ep, inv):
    def body(ep_ref, i_ref, o_ref):
        a, b = _pair(ep_ref, M, MP)
        o_ref[...] = (a + b) * i_ref[...]
    return pl.pallas_call(
        body, out_shape=jax.ShapeDtypeStruct((M, D), F32),
    )(ep, inv)


def _combine_relu_theta(vp, inv_v, w, b):
    def body(vp_ref, i_ref, w_ref, bb_ref, o_ref):
        a, b2 = _pair(vp_ref, N, NP)
        h = jnp.maximum((a + b2) * i_ref[...], 0.0)
        o_ref[...] = jnp.dot(h, w_ref[...],
                             preferred_element_type=F32) + bb_ref[...]
    return pl.pallas_call(
        body, out_shape=jax.ShapeDtypeStruct((N, D), F32),
    )(vp, inv_v, w, b.reshape(1, D))


def _final_head(vp, inv_v, wp0, bp0, wp1, bp1):
    def body(vp_ref, i_ref, w0_ref, b0_ref, w1_ref, b1_ref,
             z_ref, h_ref):
        a, b = _pair(vp_ref, N, NP)
        h = (a + b) * i_ref[...]
        h_ref[...] = h
        t = jnp.maximum(jnp.dot(h, w0_ref[...],
                                preferred_element_type=F32) + b0_ref[...], 0.0)
        z_ref[...] = jnp.dot(t, w1_ref[...],
                             preferred_element_type=F32) + b1_ref[...]
    return pl.pallas_call(
        body,
        out_shape=(
            jax.ShapeDtypeStruct((N, D), F32),
            jax.ShapeDtypeStruct((N, D), F32),
        ),
    )(vp, inv_v, wp0, bp0.reshape(1, D), wp1, bp1.reshape(1, D))


def kernel(x, node_idx, edge_idx, W0, b0, W1, b1, Wp0, bp0, Wp1, bp1):
    zrow = jnp.zeros((MROWS, D), F32)
    n2 = node_idx.reshape(-1, C)
    e2 = edge_idx.reshape(-1, C)
    iv_v2e = jnp.stack([n2, e2], axis=1).reshape(-1)
    iv_e2v = jnp.stack([e2, n2], axis=1).reshape(-1)

    h0 = _theta(x, W0, b0)
    ep, ecp, vcp = _make_seg(MP, True)(h0, iv_v2e, zrow)
    e0, inv_e, inv_v = _combine_first(
        ep,
        ecp[:M, None], ecp[MP:MP + M, None],
        vcp[:N, None], vcp[NP:NP + N, None])
    vp = _make_seg(NP)(e0, iv_e2v, zrow)

    t = _combine_relu_theta(vp, inv_v, W1, b1)
    ep2 = _make_seg(MP)(t, iv_v2e, zrow)
    e1 = _combine_scale(ep2, inv_e)
    vp2 = _make_seg(NP)(e1, iv_e2v, zrow)

    z, h = _final_head(vp2, inv_v, Wp0, bp0, Wp1, bp1)
    return (z, h)


# speedup vs baseline: 1.0109x; 1.0109x over previous
import functools

import jax
import jax.numpy as jnp
from jax import lax
from jax.experimental import pallas as pl
from jax.experimental.pallas import tpu as pltpu
from jax.experimental.pallas import tpu_sc as plsc

N = 10000
M = 5000
P = 320000
D = 128

NC, NS = 2, 16
NW = NC * NS
PPW = P // NW
C = 80
NCHUNK = PPW // C

MP = 5120
NP = 10240
MROWS = MP // NS
NROWS = NP // NS

F32 = jnp.float32

_WIN = [i * 16 for i in range(C // 16)]


@functools.cache
def _mesh():
    return plsc.VectorSubcoreMesh(
        core_axis_name="c", subcore_axis_name="s",
        num_cores=NC, num_subcores=NS)


@functools.cache
def _make_seg(acc_rows, with_counts=False):
    assert (NCHUNK - 2) % 3 == 0
    rows_per = acc_rows // NS

    out_types = [jax.ShapeDtypeStruct((NC * acc_rows, D), F32)]
    scratch = (
        [pltpu.VMEM_SHARED((acc_rows, D), F32)]
        + [pltpu.VMEM((2 * C,), jnp.int32) for _ in range(3)]
        + [pltpu.VMEM((C,), jnp.int32) for _ in range(3)]
        + [pltpu.VMEM((C, D), F32) for _ in range(3)]
        + [pltpu.SemaphoreType.DMA for _ in range(9)]
    )
    if with_counts:
        out_types += [jax.ShapeDtypeStruct((NC * MP,), F32),
                      jax.ShapeDtypeStruct((NC * NP,), F32)]
        scratch += (
            [pltpu.VMEM_SHARED((MP,), F32),
             pltpu.VMEM_SHARED((NP,), F32)]
            + [pltpu.VMEM((C,), jnp.int32) for _ in range(3)]
            + [pltpu.VMEM((C,), F32),
               pltpu.VMEM((NROWS,), F32)]
        )

    def body_fn(*refs):
        if with_counts:
            (tab, iv, zrow, out, out_ec, out_vc, acc,
             b0, b1, b2, s0, s1, s2, r0, r1, r2,
             si0, si1, si2, sg0, sg1, sg2, ss0, ss1, ss2,
             ecnt, vcnt, g0, g1, g2, ones_v, cnt_v) = refs
            gf = [g0, g1, g2]
        else:
            (tab, iv, zrow, out, acc,
             b0, b1, b2, s0, s1, s2, r0, r1, r2,
             si0, si1, si2, sg0, sg1, sg2, ss0, ss1, ss2) = refs
            gf = [None, None, None]
        buf = [b0, b1, b2]
        sf = [s0, s1, s2]
        rows = [r0, r1, r2]
        semi = [si0, si1, si2]
        semg = [sg0, sg1, sg2]
        sems = [ss0, ss1, ss2]
        cid = lax.axis_index("c")
        sid = lax.axis_index("s")
        wid = sid * NC + cid
        cbase = wid * NCHUNK

        for r in range(rows_per // MROWS):
            pltpu.sync_copy(
                zrow.at[pl.ds(0, MROWS), :],
                acc.at[pl.ds(sid * rows_per + r * MROWS, MROWS), :])
        if with_counts:
            for i in range(NROWS // 16):
                cnt_v[pl.ds(i * 16, 16)] = jnp.zeros((16,), F32)
            pltpu.sync_copy(cnt_v.at[pl.ds(0, MROWS)],
                            ecnt.at[pl.ds(sid * MROWS, MROWS)])
            pltpu.sync_copy(cnt_v, vcnt.at[pl.ds(sid * NROWS, NROWS)])
            for i in range(C // 16):
                ones_v[pl.ds(i * 16, 16)] = jnp.ones((16,), F32)
        plsc.subcore_barrier()

        def idx_start(j, s):
            pltpu.async_copy(iv.at[pl.ds((cbase + j) * 2 * C, 2 * C)],
                             buf[s], semi[s])

        def idx_wait(s):
            pltpu.make_async_copy(iv.at[pl.ds(0, 2 * C)], buf[s],
                                  semi[s]).wait()

        def extract(s):
            for w in _WIN:
                sf[s][pl.ds(w, 16)] = buf[s][pl.ds(C + w, 16)]
            if with_counts:
                for w in _WIN:
                    gf[s][pl.ds(w, 16)] = buf[s][pl.ds(w, 16)]

        def gather_start(s):
            pltpu.async_copy(tab.at[buf[s].at[pl.ds(0, C)]], rows[s],
                             semg[s])

        def gather_wait(s):
            pltpu.make_async_copy(tab.at[buf[s].at[pl.ds(0, C)]], rows[s],
                                  semg[s]).wait()

        def scatter_start(s):
            pltpu.async_copy(rows[s], acc.at[sf[s]], sems[s], add=True)
            if with_counts:
                pltpu.async_copy(ones_v, ecnt.at[sf[s]], sems[s], add=True)
                pltpu.async_copy(ones_v, vcnt.at[gf[s]], sems[s], add=True)

        def scatter_wait(s):
            pltpu.make_async_copy(rows[s], acc.at[sf[s]], sems[s]).wait()
            if with_counts:
                pltpu.make_async_copy(ones_v, ecnt.at[sf[s]],
                                      sems[s]).wait()
                pltpu.make_async_copy(ones_v, vcnt.at[gf[s]],
                                      sems[s]).wait()

        idx_start(0, 0)
        idx_start(1, 1)
        idx_start(2, 2)
        idx_wait(0)
        gather_start(0)
        idx_wait(1)
        gather_start(1)
        gather_wait(0)
        extract(0)
        idx_start(3, 0)
        scatter_start(0)

        def ring(k, carry):
            idx_wait(2)

            @pl.when(k > 0)
            def _():
                scatter_wait(2)

            gather_start(2)
            gather_wait(1)
            extract(1)
            idx_start(3 * k + 4, 1)
            scatter_start(1)
            idx_wait(0)
            scatter_wait(0)
            gather_start(0)
            gather_wait(2)
            extract(2)

            @pl.when(k < (NCHUNK - 5) // 3)
            def _():
                idx_start(3 * k + 5, 2)

            scatter_start(2)
            idx_wait(1)
            scatter_wait(1)
            gather_start(1)
            gather_wait(0)
            extract(0)

            @pl.when(k < (NCHUNK - 5) // 3)
            def _():
                idx_start(3 * k + 6, 0)

            scatter_start(0)
            return carry

        lax.fori_loop(0, (NCHUNK - 2) // 3, ring, 0)
        gather_wait(1)
        extract(1)
        scatter_start(1)
        scatter_wait(0)
        scatter_wait(1)
        scatter_wait(2)

        plsc.subcore_barrier()
        pltpu.sync_copy(
            acc.at[pl.ds(sid * rows_per, rows_per), :],
            out.at[pl.ds(cid * acc_rows + sid * rows_per, rows_per), :])
        if with_counts:
            pltpu.sync_copy(ecnt.at[pl.ds(sid * MROWS, MROWS)],
                            cnt_v.at[pl.ds(0, MROWS)])
            pltpu.sync_copy(cnt_v.at[pl.ds(0, MROWS)],
                            out_ec.at[pl.ds(cid * MP + sid * MROWS, MROWS)])
            pltpu.sync_copy(vcnt.at[pl.ds(sid * NROWS, NROWS)], cnt_v)
            pltpu.sync_copy(cnt_v,
                            out_vc.at[pl.ds(cid * NP + sid * NROWS, NROWS)])

    return functools.partial(
        pl.kernel,
        mesh=_mesh(),
        out_type=tuple(out_types) if with_counts else out_types[0],
        scratch_types=scratch,
    )(body_fn)


def _theta(x, w, b):
    def body(x_ref, w_ref, b_ref, o_ref):
        o_ref[...] = jnp.dot(x_ref[...], w_ref[...],
                             preferred_element_type=F32) + b_ref[...]
    return pl.pallas_call(
        body, out_shape=jax.ShapeDtypeStruct(x.shape, F32),
    )(x, w, b.reshape(1, D))


def _pair(ref, rows, pad_rows):
    return (ref[pl.ds(0, rows), :], ref[pl.ds(pad_rows, rows), :])


def _combine_first(ep, ec0, ec1, vc0, vc1):
    def body(ep_ref, e0_ref, e1_ref, v0_ref, v1_ref,
             eo_ref, ie_ref, iv_ref):
        ie = 1.0 / jnp.maximum(e0_ref[...] + e1_ref[...], 1.0)
        iv = 1.0 / jnp.maximum(v0_ref[...] + v1_ref[...], 1.0)
        a, b = _pair(ep_ref, M, MP)
        eo_ref[...] = (a + b) * ie
        ie_ref[...] = ie
        iv_ref[...] = iv
    return pl.pallas_call(
        body,
        out_shape=(
            jax.ShapeDtypeStruct((M, D), F32),
            jax.ShapeDtypeStruct((M, 1), F32),
            jax.ShapeDtypeStruct((N, 1), F32),
        ),
    )(ep, ec0, ec1, vc0, vc1)


def _combine_scale(ep, inv):
    def body(ep_ref, i_ref, o_ref):
        a, b = _pair(ep_ref, M, MP)
        o_ref[...] = (a + b) * i_ref[...]
    return pl.pallas_call(
        body, out_shape=jax.ShapeDtypeStruct((M, D), F32),
    )(ep, inv)


def _combine_relu_theta(vp, inv_v, w, b):
    def body(vp_ref, i_ref, w_ref, bb_ref, o_ref):
        a, b2 = _pair(vp_ref, N, NP)
        h = jnp.maximum((a + b2) * i_ref[...], 0.0)
        o_ref[...] = jnp.dot(h, w_ref[...],
                             preferred_element_type=F32) + bb_ref[...]
    return pl.pallas_call(
        body, out_shape=jax.ShapeDtypeStruct((N, D), F32),
    )(vp, inv_v, w, b.reshape(1, D))


def _final_head(vp, inv_v, wp0, bp0, wp1, bp1):
    def body(vp_ref, i_ref, w0_ref, b0_ref, w1_ref, b1_ref,
             z_ref, h_ref):
        a, b = _pair(vp_ref, N, NP)
        h = (a + b) * i_ref[...]
        h_ref[...] = h
        t = jnp.maximum(jnp.dot(h, w0_ref[...],
                                preferred_element_type=F32) + b0_ref[...], 0.0)
        z_ref[...] = jnp.dot(t, w1_ref[...],
                             preferred_element_type=F32) + b1_ref[...]
    return pl.pallas_call(
        body,
        out_shape=(
            jax.ShapeDtypeStruct((N, D), F32),
            jax.ShapeDtypeStruct((N, D), F32),
        ),
    )(vp, inv_v, wp0, bp0.reshape(1, D), wp1, bp1.reshape(1, D))


def kernel(x, node_idx, edge_idx, W0, b0, W1, b1, Wp0, bp0, Wp1, bp1):
    zrow = jnp.zeros((MROWS, D), F32)
    n2 = node_idx.reshape(-1, C)
    e2 = edge_idx.reshape(-1, C)
    iv_v2e = jnp.stack([n2, e2], axis=1).reshape(-1)
    iv_e2v = jnp.stack([e2, n2], axis=1).reshape(-1)

    h0 = _theta(x, W0, b0)
    ep, ecp, vcp = _make_seg(MP, True)(h0, iv_v2e, zrow)
    e0, inv_e, inv_v = _combine_first(
        ep,
        ecp[:M, None], ecp[MP:MP + M, None],
        vcp[:N, None], vcp[NP:NP + N, None])
    vp = _make_seg(NP)(e0, iv_e2v, zrow)

    t = _combine_relu_theta(vp, inv_v, W1, b1)
    ep2 = _make_seg(MP)(t, iv_v2e, zrow)
    e1 = _combine_scale(ep2, inv_e)
    vp2 = _make_seg(NP)(e1, iv_e2v, zrow)

    z, h = _final_head(vp2, inv_v, Wp0, bp0, Wp1, bp1)
    return (z, h)
